# baseline (speedup 1.0000x reference)
import functools

import jax
import jax.numpy as jnp
from jax import lax
from jax.experimental import pallas as pl
from jax.experimental.pallas import tpu as pltpu
from jax.experimental.pallas import tpu_sc as plsc

TOP_P = 0.8
B = 64
N = 100000
SLACK = 512
NREC = N + SLACK
W = 2048
NFULL = 48
TAIL = N - NFULL * W
TAILCH = TAIL // 16
RADIX = 32768
KMAX = (1 << 30) - 1
PAD_N = 100352
ROWS2D = 784


def _sc_sort(kin_flat):
    mesh = plsc.VectorSubcoreMesh(
        core_axis_name="c", subcore_axis_name="s", num_cores=2,
        num_subcores=16)

    @functools.partial(
        pl.kernel,
        out_type=[
            jax.ShapeDtypeStruct((B * NREC, 16), jnp.int32),
            jax.ShapeDtypeStruct((B * NREC, 16), jnp.int32),
        ],
        mesh=mesh,
        scratch_types=[
            pltpu.VMEM((RADIX,), jnp.int32),
            pltpu.VMEM((W,), jnp.int32),
            pltpu.VMEM((W, 16), jnp.int32),
            pltpu.VMEM((W, 16), jnp.int32),
            pltpu.VMEM((W,), jnp.int32),
            pltpu.SemaphoreType.DMA,
        ],
        compiler_params=pltpu.CompilerParams(
            needs_layout_passes=False, use_tc_tiling_on_sc=False),
    )
    def sortk(kin, rec0, rec1, hist, win0, win1, stg, didx, sem):
        wid = lax.axis_index("s") * 2 + lax.axis_index("c")
        iota = lax.iota(jnp.int32, 16)
        zeros16 = jnp.zeros((16,), jnp.int32)
        ones16 = jnp.ones((16,), jnp.int32)

        def zero_hist():
            def zb(i, carry):
                hist[pl.ds(i * 16, 16)] = zeros16
                return carry
            lax.fori_loop(0, RADIX // 16, zb, jnp.int32(0))

        def excl_scan():
            def sb(i, carry):
                v = hist[pl.ds(i * 16, 16)]
                c = plsc.cumsum(v)
                hist[pl.ds(i * 16, 16)] = c - v + carry
                return carry + jnp.max(c)
            lax.fori_loop(0, RADIX // 16, sb, jnp.int32(0))

        def hist_chunk(d):
            counts, lastm = plsc.scan_count(d)
            plsc.addupdate_scatter(hist, [d], counts, mask=lastm)

        def run_pass(row, is_pass0):
            rbrec = row * NREC
            if is_pass0:
                src_off = row * N
                recdst = rec0
            else:
                src_off = rbrec
                recdst = rec1

            def digit(kk):
                if is_pass0:
                    return kk & jnp.int32(0x7FFF)
                return lax.shift_right_logical(kk, jnp.int32(15))

            def copy_in(w, nelems):
                if is_pass0:
                    pltpu.sync_copy(
                        kin.at[pl.ds(src_off + w * W, nelems)],
                        win0.at[pl.ds(0, nelems)])
                else:
                    pltpu.sync_copy(
                        rec0.at[pl.ds(src_off + w * W, nelems)],
                        win1.at[pl.ds(0, nelems)])

            def get_kv(c, w):
                l = c * 16 + iota
                if is_pass0:
                    kk = win0[pl.ds(c * 16, 16)]
                    vv = w * W + l
                else:
                    kk = plsc.load_gather(win1, [l, zeros16])
                    vv = plsc.load_gather(win1, [l, ones16])
                return kk, vv, l

            zero_hist()

            def hw_body(w, carry):
                copy_in(w, W)
                def hc(c, cc):
                    kk, _, _ = get_kv(c, w)
                    hist_chunk(digit(kk))
                    return cc
                lax.fori_loop(0, W // 16, hc, jnp.int32(0))
                return carry
            lax.fori_loop(0, NFULL, hw_body, jnp.int32(0))
            copy_in(jnp.int32(NFULL), TAIL)
            def hct(c, cc):
                kk, _, _ = get_kv(c, jnp.int32(NFULL))
                hist_chunk(digit(kk))
                return cc
            lax.fori_loop(0, TAILCH, hct, jnp.int32(0))

            excl_scan()

            def stage_chunk(c, w):
                kk, vv, l = get_kv(c, w)
                d = digit(kk)
                counts, lastm = plsc.scan_count(d)
                base = plsc.load_gather(hist, [d])
                dest = base + counts - 1 + rbrec
                dest = jnp.clip(dest, 0, jnp.int32(B * NREC - 1))
                didx[pl.ds(c * 16, 16)] = dest
                plsc.store_scatter(stg, [l, zeros16], kk)
                plsc.store_scatter(stg, [l, ones16], vv)
                plsc.addupdate_scatter(hist, [d], counts, mask=lastm)

            def flush():
                pltpu.async_copy(stg, recdst.at[didx], sem).wait()

            def pw_body(w, carry):
                copy_in(w, W)
                def pc(c, cc):
                    stage_chunk(c, w)
                    return cc
                lax.fori_loop(0, W // 16, pc, jnp.int32(0))
                flush()
                return carry
            lax.fori_loop(0, NFULL, pw_body, jnp.int32(0))

            copy_in(jnp.int32(NFULL), TAIL)
            def pct(c, cc):
                stage_chunk(c, jnp.int32(NFULL))
                return cc
            lax.fori_loop(0, TAILCH, pct, jnp.int32(0))
            def padb(c, cc):
                l = c * 16 + iota
                didx[pl.ds(c * 16, 16)] = rbrec + N + (l - TAIL)
                return cc
            lax.fori_loop(TAILCH, W // 16, padb, jnp.int32(0))
            flush()

        def row_body(rr, carry):
            row = wid * 2 + rr
            run_pass(row, True)
            run_pass(row, False)
            return carry
        lax.fori_loop(0, 2, row_body, jnp.int32(0))

    return sortk(kin_flat)


def _topp_body(sp_ref, sidx_ref, g_ref, out_ref, tok_ref):
    p = sp_ref[0]
    sidx = sidx_ref[0]
    g = g_ref[0]

    r1 = lax.broadcasted_iota(jnp.int32, (128, 128), 0)
    c1 = lax.broadcasted_iota(jnp.int32, (128, 128), 1)
    t1 = (r1 <= c1).astype(jnp.float32)
    cum_lane = lax.dot_general(
        p, t1, (((1,), (0,)), ((), ())), precision=lax.Precision.HIGHEST)
    s_col = cum_lane[:, 127:128]
    r2 = lax.broadcasted_iota(jnp.int32, (ROWS2D, ROWS2D), 0)
    c2 = lax.broadcasted_iota(jnp.int32, (ROWS2D, ROWS2D), 1)
    t2 = (c2 < r2).astype(jnp.float32)
    carr = lax.dot_general(
        t2, s_col, (((1,), (0,)), ((), ())), precision=lax.Precision.HIGHEST)
    cum = cum_lane + carr

    keep = (cum - p) <= TOP_P
    pk = jnp.where(keep, p, 0.0)
    s = jnp.sum(pk)
    out = pk / s
    out_ref[0] = out

    t = jnp.log(out + 1e-20) + g
    m = jnp.max(t)
    pos = lax.broadcasted_iota(jnp.int32, t.shape, 0) * 128 + \
        lax.broadcasted_iota(jnp.int32, t.shape, 1)
    big = jnp.int32(2**31 - 1)
    jmin = jnp.min(jnp.where(t == m, pos, big))
    tok = jnp.max(jnp.where(pos == jmin, sidx, jnp.int32(-1)))
    tok_ref[0] = jnp.full((1, 128), tok, jnp.int32)


def kernel(logits):
    b, n = logits.shape
    probs = jax.nn.softmax(logits, axis=-1)
    kbits = lax.bitcast_convert_type(probs, jnp.int32)
    kin = (KMAX - kbits).reshape(-1)

    _, rec1 = _sc_sort(kin)
    rec = rec1.reshape(B, NREC, 16)[:, :N, :2]
    skey = rec[..., 0]
    sidx = rec[..., 1]
    sp = lax.bitcast_convert_type(KMAX - skey, jnp.float32)

    g = jax.random.gumbel(jax.random.key(42), (b, n), jnp.float32)

    pad = PAD_N - n
    spp = jnp.pad(sp, ((0, 0), (0, pad))).reshape(b, ROWS2D, 128)
    sip = jnp.pad(sidx, ((0, 0), (0, pad))).reshape(b, ROWS2D, 128)
    gg = jnp.pad(g, ((0, 0), (0, pad))).reshape(b, ROWS2D, 128)

    out, tok = pl.pallas_call(
        _topp_body,
        grid=(b,),
        in_specs=[
            pl.BlockSpec((1, ROWS2D, 128), lambda i: (i, 0, 0)),
            pl.BlockSpec((1, ROWS2D, 128), lambda i: (i, 0, 0)),
            pl.BlockSpec((1, ROWS2D, 128), lambda i: (i, 0, 0)),
        ],
        out_specs=[
            pl.BlockSpec((1, ROWS2D, 128), lambda i: (i, 0, 0)),
            pl.BlockSpec((1, 1, 128), lambda i: (i, 0, 0)),
        ],
        out_shape=[
            jax.ShapeDtypeStruct((b, ROWS2D, 128), jnp.float32),
            jax.ShapeDtypeStruct((b, 1, 128), jnp.int32),
        ],
    )(spp, sip, gg)

    sorted_probs_out = out.reshape(b, PAD_N)[:, :n]
    return (tok[:, 0, :1], sorted_probs_out)


# speedup vs baseline: 3.2938x; 3.2938x over previous
import functools

import jax
import jax.numpy as jnp
from jax import lax
from jax.experimental import pallas as pl
from jax.experimental.pallas import tpu as pltpu
from jax.experimental.pallas import tpu_sc as plsc

TOP_P = 0.8
B = 64
N = 100000
SLACK = 512
NREC = N + SLACK
W = 2048
NFULL = 48
TAIL = N - NFULL * W
TAILCH = TAIL // 16
RADIX = 32768
KMAX = (1 << 30) - 1
PAD_N = 100352
ROWS2D = 784


def _sc_sort(kin_flat):
    mesh = plsc.VectorSubcoreMesh(
        core_axis_name="c", subcore_axis_name="s", num_cores=2,
        num_subcores=16)

    @functools.partial(
        pl.kernel,
        out_type=[
            jax.ShapeDtypeStruct((B * N,), jnp.int32),
            jax.ShapeDtypeStruct((B * N,), jnp.int32),
        ],
        mesh=mesh,
        scratch_types=[
            pltpu.HBM((B * NREC, 16), jnp.int32),
            pltpu.HBM((B * NREC, 16), jnp.int32),
            pltpu.VMEM((RADIX,), jnp.int32),
            pltpu.VMEM((W,), jnp.int32),
            pltpu.VMEM((W, 16), jnp.int32),
            pltpu.VMEM((W, 16), jnp.int32),
            pltpu.VMEM((W,), jnp.int32),
            pltpu.VMEM((W,), jnp.int32),
            pltpu.VMEM((W,), jnp.int32),
            pltpu.SemaphoreType.DMA,
        ],
        compiler_params=pltpu.CompilerParams(
            needs_layout_passes=False, use_tc_tiling_on_sc=False),
    )
    def sortk(kin, skey_out, sidx_out, rec0, rec1, hist, win0, win1,
              stg, didx, ckey, cidx, sem):
        wid = lax.axis_index("s") * 2 + lax.axis_index("c")
        iota = lax.iota(jnp.int32, 16)
        zeros16 = jnp.zeros((16,), jnp.int32)
        ones16 = jnp.ones((16,), jnp.int32)

        def zero_hist():
            def zb(i, carry):
                hist[pl.ds(i * 16, 16)] = zeros16
                return carry
            lax.fori_loop(0, RADIX // 16, zb, jnp.int32(0))

        def excl_scan():
            def sb(i, carry):
                v = hist[pl.ds(i * 16, 16)]
                c = plsc.cumsum(v)
                hist[pl.ds(i * 16, 16)] = c - v + carry
                return carry + jnp.max(c)
            lax.fori_loop(0, RADIX // 16, sb, jnp.int32(0))

        def hist_chunk(d):
            counts, lastm = plsc.scan_count(d)
            plsc.addupdate_scatter(hist, [d], counts, mask=lastm)

        def run_pass(row, is_pass0):
            rbrec = row * NREC
            if is_pass0:
                src_off = row * N
                recdst = rec0
            else:
                src_off = rbrec
                recdst = rec1

            def digit(kk):
                if is_pass0:
                    return kk & jnp.int32(0x7FFF)
                return lax.shift_right_logical(kk, jnp.int32(15))

            def copy_in(w, nelems):
                if is_pass0:
                    pltpu.sync_copy(
                        kin.at[pl.ds(src_off + w * W, nelems)],
                        win0.at[pl.ds(0, nelems)])
                else:
                    pltpu.sync_copy(
                        rec0.at[pl.ds(src_off + w * W, nelems)],
                        win1.at[pl.ds(0, nelems)])

            def get_kv(c, w):
                l = c * 16 + iota
                if is_pass0:
                    kk = win0[pl.ds(c * 16, 16)]
                    vv = w * W + l
                else:
                    kk = plsc.load_gather(win1, [l, zeros16])
                    vv = plsc.load_gather(win1, [l, ones16])
                return kk, vv, l

            zero_hist()

            def hw_body(w, carry):
                copy_in(w, W)
                def hc(c, cc):
                    kk, _, _ = get_kv(c, w)
                    hist_chunk(digit(kk))
                    return cc
                lax.fori_loop(0, W // 16, hc, jnp.int32(0))
                return carry
            lax.fori_loop(0, NFULL, hw_body, jnp.int32(0))
            copy_in(jnp.int32(NFULL), TAIL)
            def hct(c, cc):
                kk, _, _ = get_kv(c, jnp.int32(NFULL))
                hist_chunk(digit(kk))
                return cc
            lax.fori_loop(0, TAILCH, hct, jnp.int32(0))

            excl_scan()

            def stage_chunk(c, w):
                kk, vv, l = get_kv(c, w)
                d = digit(kk)
                counts, lastm = plsc.scan_count(d)
                base = plsc.load_gather(hist, [d])
                dest = base + counts - 1 + rbrec
                dest = jnp.clip(dest, 0, jnp.int32(B * NREC - 1))
                didx[pl.ds(c * 16, 16)] = dest
                plsc.store_scatter(stg, [l, zeros16], kk)
                plsc.store_scatter(stg, [l, ones16], vv)
                plsc.addupdate_scatter(hist, [d], counts, mask=lastm)

            def flush():
                pltpu.async_copy(stg, recdst.at[didx], sem).wait()

            def pw_body(w, carry):
                copy_in(w, W)
                def pc(c, cc):
                    stage_chunk(c, w)
                    return cc
                lax.fori_loop(0, W // 16, pc, jnp.int32(0))
                flush()
                return carry
            lax.fori_loop(0, NFULL, pw_body, jnp.int32(0))

            copy_in(jnp.int32(NFULL), TAIL)
            def pct(c, cc):
                stage_chunk(c, jnp.int32(NFULL))
                return cc
            lax.fori_loop(0, TAILCH, pct, jnp.int32(0))
            def padb(c, cc):
                l = c * 16 + iota
                didx[pl.ds(c * 16, 16)] = rbrec + N + (l - TAIL)
                return cc
            lax.fori_loop(TAILCH, W // 16, padb, jnp.int32(0))
            flush()

        def compact(row):
            rbrec = row * NREC
            out_off = row * N

            def cw(w, nelems):
                pltpu.sync_copy(
                    rec1.at[pl.ds(rbrec + w * W, nelems)],
                    win1.at[pl.ds(0, nelems)])
                def cb(c, cc):
                    l = c * 16 + iota
                    kk = plsc.load_gather(win1, [l, zeros16])
                    vv = plsc.load_gather(win1, [l, ones16])
                    ckey[pl.ds(c * 16, 16)] = kk
                    cidx[pl.ds(c * 16, 16)] = vv
                    return cc
                lax.fori_loop(0, nelems // 16, cb, jnp.int32(0))
                pltpu.sync_copy(
                    ckey.at[pl.ds(0, nelems)],
                    skey_out.at[pl.ds(out_off + w * W, nelems)])
                pltpu.sync_copy(
                    cidx.at[pl.ds(0, nelems)],
                    sidx_out.at[pl.ds(out_off + w * W, nelems)])

            def cwb(w, carry):
                cw(w, W)
                return carry
            lax.fori_loop(0, NFULL, cwb, jnp.int32(0))
            cw(jnp.int32(NFULL), TAIL)

        def row_body(rr, carry):
            row = wid * 2 + rr
            run_pass(row, True)
            run_pass(row, False)
            compact(row)
            return carry
        lax.fori_loop(0, 2, row_body, jnp.int32(0))

    return sortk(kin_flat)


def _topp_body(sp_ref, sidx_ref, g_ref, out_ref, tok_ref):
    p = sp_ref[0]
    sidx = sidx_ref[0]
    g = g_ref[0]

    r1 = lax.broadcasted_iota(jnp.int32, (128, 128), 0)
    c1 = lax.broadcasted_iota(jnp.int32, (128, 128), 1)
    t1 = (r1 <= c1).astype(jnp.float32)
    cum_lane = lax.dot_general(
        p, t1, (((1,), (0,)), ((), ())), precision=lax.Precision.HIGHEST)
    s_col = cum_lane[:, 127:128]
    r2 = lax.broadcasted_iota(jnp.int32, (ROWS2D, ROWS2D), 0)
    c2 = lax.broadcasted_iota(jnp.int32, (ROWS2D, ROWS2D), 1)
    t2 = (c2 < r2).astype(jnp.float32)
    carr = lax.dot_general(
        t2, s_col, (((1,), (0,)), ((), ())), precision=lax.Precision.HIGHEST)
    cum = cum_lane + carr

    keep = (cum - p) <= TOP_P
    pk = jnp.where(keep, p, 0.0)
    s = jnp.sum(pk)
    out = pk / s
    out_ref[0] = out

    t = jnp.log(out + 1e-20) + g
    m = jnp.max(t)
    pos = lax.broadcasted_iota(jnp.int32, t.shape, 0) * 128 + \
        lax.broadcasted_iota(jnp.int32, t.shape, 1)
    big = jnp.int32(2**31 - 1)
    jmin = jnp.min(jnp.where(t == m, pos, big))
    tok = jnp.max(jnp.where(pos == jmin, sidx, jnp.int32(-1)))
    tok_ref[0] = jnp.full((1, 128), tok, jnp.int32)


def kernel(logits):
    b, n = logits.shape
    probs = jax.nn.softmax(logits, axis=-1)
    kbits = lax.bitcast_convert_type(probs, jnp.int32)
    kin = (KMAX - kbits).reshape(-1)

    skey, sidx = _sc_sort(kin)
    skey = skey.reshape(B, N)
    sidx = sidx.reshape(B, N)
    sp = lax.bitcast_convert_type(KMAX - skey, jnp.float32)

    g = jax.random.gumbel(jax.random.key(42), (b, n), jnp.float32)

    pad = PAD_N - n
    spp = jnp.pad(sp, ((0, 0), (0, pad))).reshape(b, ROWS2D, 128)
    sip = jnp.pad(sidx, ((0, 0), (0, pad))).reshape(b, ROWS2D, 128)
    gg = jnp.pad(g, ((0, 0), (0, pad))).reshape(b, ROWS2D, 128)

    out, tok = pl.pallas_call(
        _topp_body,
        grid=(b,),
        in_specs=[
            pl.BlockSpec((1, ROWS2D, 128), lambda i: (i, 0, 0)),
            pl.BlockSpec((1, ROWS2D, 128), lambda i: (i, 0, 0)),
            pl.BlockSpec((1, ROWS2D, 128), lambda i: (i, 0, 0)),
        ],
        out_specs=[
            pl.BlockSpec((1, ROWS2D, 128), lambda i: (i, 0, 0)),
            pl.BlockSpec((1, 1, 128), lambda i: (i, 0, 0)),
        ],
        out_shape=[
            jax.ShapeDtypeStruct((b, ROWS2D, 128), jnp.float32),
            jax.ShapeDtypeStruct((b, 1, 128), jnp.int32),
        ],
    )(spp, sip, gg)

    sorted_probs_out = out.reshape(b, PAD_N)[:, :n]
    return (tok[:, 0, :1], sorted_probs_out)


# speedup vs baseline: 3.4880x; 1.0590x over previous
import functools

import jax
import jax.numpy as jnp
from jax import lax
from jax.experimental import pallas as pl
from jax.experimental.pallas import tpu as pltpu
from jax.experimental.pallas import tpu_sc as plsc

TOP_P = 0.8
B = 64
N = 100000
SLACK = 512
NREC = N + SLACK
W = 2048
NFULL = 48
TAIL = N - NFULL * W
TAILCH = TAIL // 16
RADIX = 32768
KMAX = (1 << 30) - 1
PAD_N = 100352
ROWS2D = 784


def _sc_sort(kin_flat):
    mesh = plsc.VectorSubcoreMesh(
        core_axis_name="c", subcore_axis_name="s", num_cores=2,
        num_subcores=16)

    @functools.partial(
        pl.kernel,
        out_type=[
            jax.ShapeDtypeStruct((B * N,), jnp.int32),
            jax.ShapeDtypeStruct((B * N,), jnp.int32),
        ],
        mesh=mesh,
        scratch_types=[
            pltpu.HBM((B * NREC, 16), jnp.int32),
            pltpu.HBM((B * NREC, 16), jnp.int32),
            pltpu.VMEM((RADIX,), jnp.int32),
            pltpu.VMEM((W,), jnp.int32),
            pltpu.VMEM((W, 16), jnp.int32),
            pltpu.VMEM((W, 16), jnp.int32),
            pltpu.VMEM((W,), jnp.int32),
            pltpu.VMEM((W,), jnp.int32),
            pltpu.VMEM((W,), jnp.int32),
            pltpu.SemaphoreType.DMA,
        ],
        compiler_params=pltpu.CompilerParams(
            needs_layout_passes=False, use_tc_tiling_on_sc=False),
    )
    def sortk(kin, skey_out, sidx_out, rec0, rec1, hist, win0, win1,
              stg, didx, ckey, cidx, sem):
        wid = lax.axis_index("s") * 2 + lax.axis_index("c")
        iota = lax.iota(jnp.int32, 16)
        zeros16 = jnp.zeros((16,), jnp.int32)
        ones16 = jnp.ones((16,), jnp.int32)

        def zero_hist():
            def zb(i, carry):
                hist[pl.ds(i * 16, 16)] = zeros16
                return carry
            lax.fori_loop(0, RADIX // 16, zb, jnp.int32(0))

        def excl_scan():
            def sb(i, carry):
                v = hist[pl.ds(i * 16, 16)]
                c = plsc.cumsum(v)
                hist[pl.ds(i * 16, 16)] = c - v + carry
                return carry + jnp.max(c)
            lax.fori_loop(0, RADIX // 16, sb, jnp.int32(0))

        def hist_chunk(d):
            counts, lastm = plsc.scan_count(d)
            plsc.addupdate_scatter(hist, [d], counts, mask=lastm)

        def run_pass(row, is_pass0):
            rbrec = row * NREC
            if is_pass0:
                src_off = row * N
                recdst = rec0
            else:
                src_off = rbrec
                recdst = rec1

            def digit(kk):
                if is_pass0:
                    return kk & jnp.int32(0x7FFF)
                return lax.shift_right_logical(kk, jnp.int32(15))

            def copy_in(w, nelems):
                if is_pass0:
                    pltpu.sync_copy(
                        kin.at[pl.ds(src_off + w * W, nelems)],
                        win0.at[pl.ds(0, nelems)])
                else:
                    pltpu.sync_copy(
                        rec0.at[pl.ds(src_off + w * W, nelems)],
                        win1.at[pl.ds(0, nelems)])

            def get_kv(c, w):
                l = c * 16 + iota
                if is_pass0:
                    kk = win0[pl.ds(c * 16, 16)]
                    vv = w * W + l
                else:
                    kk = plsc.load_gather(win1, [l, zeros16])
                    vv = plsc.load_gather(win1, [l, ones16])
                return kk, vv, l

            zero_hist()

            def hist_in(w, nelems):
                pltpu.sync_copy(
                    kin.at[pl.ds(row * N + w * W, nelems)],
                    win0.at[pl.ds(0, nelems)])

            def hw_body(w, carry):
                hist_in(w, W)
                def hc(c, cc):
                    kk = win0[pl.ds(c * 16, 16)]
                    hist_chunk(digit(kk))
                    return cc
                lax.fori_loop(0, W // 16, hc, jnp.int32(0))
                return carry
            lax.fori_loop(0, NFULL, hw_body, jnp.int32(0))
            hist_in(jnp.int32(NFULL), TAIL)
            def hct(c, cc):
                kk = win0[pl.ds(c * 16, 16)]
                hist_chunk(digit(kk))
                return cc
            lax.fori_loop(0, TAILCH, hct, jnp.int32(0))

            excl_scan()

            def stage_chunk(c, w):
                kk, vv, l = get_kv(c, w)
                d = digit(kk)
                counts, lastm = plsc.scan_count(d)
                base = plsc.load_gather(hist, [d])
                dest = base + counts - 1 + rbrec
                dest = jnp.clip(dest, 0, jnp.int32(B * NREC - 1))
                didx[pl.ds(c * 16, 16)] = dest
                plsc.store_scatter(stg, [l, zeros16], kk)
                plsc.store_scatter(stg, [l, ones16], vv)
                plsc.addupdate_scatter(hist, [d], counts, mask=lastm)

            def flush():
                pltpu.async_copy(stg, recdst.at[didx], sem).wait()

            def pw_body(w, carry):
                copy_in(w, W)
                def pc(c, cc):
                    stage_chunk(c, w)
                    return cc
                lax.fori_loop(0, W // 16, pc, jnp.int32(0))
                flush()
                return carry
            lax.fori_loop(0, NFULL, pw_body, jnp.int32(0))

            copy_in(jnp.int32(NFULL), TAIL)
            def pct(c, cc):
                stage_chunk(c, jnp.int32(NFULL))
                return cc
            lax.fori_loop(0, TAILCH, pct, jnp.int32(0))
            def padb(c, cc):
                l = c * 16 + iota
                didx[pl.ds(c * 16, 16)] = rbrec + N + (l - TAIL)
                return cc
            lax.fori_loop(TAILCH, W // 16, padb, jnp.int32(0))
            flush()

        def compact(row):
            rbrec = row * NREC
            out_off = row * N

            def cw(w, nelems):
                pltpu.sync_copy(
                    rec1.at[pl.ds(rbrec + w * W, nelems)],
                    win1.at[pl.ds(0, nelems)])
                def cb(c, cc):
                    l = c * 16 + iota
                    kk = plsc.load_gather(win1, [l, zeros16])
                    vv = plsc.load_gather(win1, [l, ones16])
                    ckey[pl.ds(c * 16, 16)] = kk
                    cidx[pl.ds(c * 16, 16)] = vv
                    return cc
                lax.fori_loop(0, nelems // 16, cb, jnp.int32(0))
                pltpu.sync_copy(
                    ckey.at[pl.ds(0, nelems)],
                    skey_out.at[pl.ds(out_off + w * W, nelems)])
                pltpu.sync_copy(
                    cidx.at[pl.ds(0, nelems)],
                    sidx_out.at[pl.ds(out_off + w * W, nelems)])

            def cwb(w, carry):
                cw(w, W)
                return carry
            lax.fori_loop(0, NFULL, cwb, jnp.int32(0))
            cw(jnp.int32(NFULL), TAIL)

        def row_body(rr, carry):
            row = wid * 2 + rr
            run_pass(row, True)
            run_pass(row, False)
            compact(row)
            return carry
        lax.fori_loop(0, 2, row_body, jnp.int32(0))

    return sortk(kin_flat)


def _topp_body(sp_ref, sidx_ref, g_ref, out_ref, tok_ref):
    p = sp_ref[0]
    sidx = sidx_ref[0]
    g = g_ref[0]

    r1 = lax.broadcasted_iota(jnp.int32, (128, 128), 0)
    c1 = lax.broadcasted_iota(jnp.int32, (128, 128), 1)
    t1 = (r1 <= c1).astype(jnp.float32)
    cum_lane = lax.dot_general(
        p, t1, (((1,), (0,)), ((), ())), precision=lax.Precision.HIGHEST)
    s_col = cum_lane[:, 127:128]
    r2 = lax.broadcasted_iota(jnp.int32, (ROWS2D, ROWS2D), 0)
    c2 = lax.broadcasted_iota(jnp.int32, (ROWS2D, ROWS2D), 1)
    t2 = (c2 < r2).astype(jnp.float32)
    carr = lax.dot_general(
        t2, s_col, (((1,), (0,)), ((), ())), precision=lax.Precision.HIGHEST)
    cum = cum_lane + carr

    keep = (cum - p) <= TOP_P
    pk = jnp.where(keep, p, 0.0)
    s = jnp.sum(pk)
    out = pk / s
    out_ref[0] = out

    t = jnp.log(out + 1e-20) + g
    m = jnp.max(t)
    pos = lax.broadcasted_iota(jnp.int32, t.shape, 0) * 128 + \
        lax.broadcasted_iota(jnp.int32, t.shape, 1)
    big = jnp.int32(2**31 - 1)
    jmin = jnp.min(jnp.where(t == m, pos, big))
    tok = jnp.max(jnp.where(pos == jmin, sidx, jnp.int32(-1)))
    tok_ref[0] = jnp.full((1, 128), tok, jnp.int32)


def kernel(logits):
    b, n = logits.shape
    probs = jax.nn.softmax(logits, axis=-1)
    kbits = lax.bitcast_convert_type(probs, jnp.int32)
    kin = (KMAX - kbits).reshape(-1)

    skey, sidx = _sc_sort(kin)
    skey = skey.reshape(B, N)
    sidx = sidx.reshape(B, N)
    sp = lax.bitcast_convert_type(KMAX - skey, jnp.float32)

    g = jax.random.gumbel(jax.random.key(42), (b, n), jnp.float32)

    pad = PAD_N - n
    spp = jnp.pad(sp, ((0, 0), (0, pad))).reshape(b, ROWS2D, 128)
    sip = jnp.pad(sidx, ((0, 0), (0, pad))).reshape(b, ROWS2D, 128)
    gg = jnp.pad(g, ((0, 0), (0, pad))).reshape(b, ROWS2D, 128)

    out, tok = pl.pallas_call(
        _topp_body,
        grid=(b,),
        in_specs=[
            pl.BlockSpec((1, ROWS2D, 128), lambda i: (i, 0, 0)),
            pl.BlockSpec((1, ROWS2D, 128), lambda i: (i, 0, 0)),
            pl.BlockSpec((1, ROWS2D, 128), lambda i: (i, 0, 0)),
        ],
        out_specs=[
            pl.BlockSpec((1, ROWS2D, 128), lambda i: (i, 0, 0)),
            pl.BlockSpec((1, 1, 128), lambda i: (i, 0, 0)),
        ],
        out_shape=[
            jax.ShapeDtypeStruct((b, ROWS2D, 128), jnp.float32),
            jax.ShapeDtypeStruct((b, 1, 128), jnp.int32),
        ],
    )(spp, sip, gg)

    sorted_probs_out = out.reshape(b, PAD_N)[:, :n]
    return (tok[:, 0, :1], sorted_probs_out)
